# baseline (speedup 1.0000x reference)
import functools

import jax
import jax.numpy as jnp
from jax import lax
from jax.experimental import pallas as pl
from jax.experimental.pallas import tpu as pltpu
from jax.experimental.pallas import tpu_sc as plsc

LANES = 16
CHUNK = 128


def _sc_dot_kernel(batch, n_factors, n_workers, nc):
    bpw = batch // n_workers
    nch = bpw // CHUNK
    ngrp = bpw // LANES

    mesh = plsc.VectorSubcoreMesh(core_axis_name="c", subcore_axis_name="s")

    @functools.partial(
        pl.kernel,
        mesh=mesh,
        out_type=jax.ShapeDtypeStruct((batch,), jnp.float32),
        compiler_params=pltpu.CompilerParams(
            needs_layout_passes=False, use_tc_tiling_on_sc=False),
        scratch_types=[
            pltpu.VMEM((nch, CHUNK), jnp.int32),
            pltpu.VMEM((nch, CHUNK), jnp.int32),
            pltpu.VMEM((bpw, n_factors), jnp.float32),
            pltpu.VMEM((bpw, n_factors), jnp.float32),
            pltpu.VMEM((bpw,), jnp.float32),
            pltpu.SemaphoreType.DMA,
        ],
    )
    def sc_kernel(u_hbm, i_hbm, uvec_hbm, ivec_hbm, out_hbm,
                  uidx_v, iidx_v, urows_v, irows_v, out_v, sem):
        wid = lax.axis_index("s") * nc + lax.axis_index("c")
        base = wid * bpw

        for j in range(nch):
            pltpu.sync_copy(u_hbm.at[pl.ds(base + j * CHUNK, CHUNK)],
                            uidx_v.at[j])
            pltpu.sync_copy(i_hbm.at[pl.ds(base + j * CHUNK, CHUNK)],
                            iidx_v.at[j])

        copies = []
        for j in range(nch):
            copies.append(pltpu.async_copy(
                uvec_hbm.at[uidx_v.at[j]],
                urows_v.at[pl.ds(j * CHUNK, CHUNK)], sem))
            copies.append(pltpu.async_copy(
                ivec_hbm.at[iidx_v.at[j]],
                irows_v.at[pl.ds(j * CHUNK, CHUNK)], sem))
        for c in copies:
            c.wait()

        nseg = n_factors // LANES
        unroll = 4
        lane_iota = lax.broadcasted_iota(jnp.int32, (LANES,), 0)
        last_lane = lane_iota == (LANES - 1)

        def rows(r0, _):
            for k in range(unroll):
                r = r0 * unroll + k
                acc = (urows_v[r, pl.ds(0, LANES)]
                       * irows_v[r, pl.ds(0, LANES)])
                for t in range(1, nseg):
                    acc = acc + (urows_v[r, pl.ds(t * LANES, LANES)]
                                 * irows_v[r, pl.ds(t * LANES, LANES)])
                tot = plsc.cumsum(acc)
                plsc.store_scatter(out_v, [jnp.full((LANES,), r, jnp.int32)],
                                   tot, mask=last_lane)
            return _

        lax.fori_loop(0, bpw // unroll, rows, None)

        pltpu.sync_copy(out_v, out_hbm.at[pl.ds(base, bpw)])

    return sc_kernel


def kernel(u, i, user_vec, item_vec):
    batch = u.shape[0]
    n_factors = user_vec.shape[1]
    info = plsc.get_sparse_core_info()
    nc, ns = info.num_cores, info.num_subcores
    n_workers = nc * ns
    fn = _sc_dot_kernel(batch, n_factors, n_workers, nc)
    return fn(u, i, user_vec, item_vec)


# speedup vs baseline: 1.5476x; 1.5476x over previous
import functools

import jax
import jax.numpy as jnp
from jax import lax
from jax.experimental import pallas as pl
from jax.experimental.pallas import tpu as pltpu
from jax.experimental.pallas import tpu_sc as plsc

LANES = 16
CHUNK = 128
GROUPS = CHUNK // LANES


def _sc_dot_kernel(batch, n_factors, n_workers, nc):
    bpw = batch // n_workers
    nch = bpw // CHUNK

    mesh = plsc.VectorSubcoreMesh(core_axis_name="c", subcore_axis_name="s")

    @functools.partial(
        pl.kernel,
        mesh=mesh,
        out_type=jax.ShapeDtypeStruct((batch,), jnp.float32),
        compiler_params=pltpu.CompilerParams(needs_layout_passes=False),
        scratch_types=[
            pltpu.VMEM((bpw,), jnp.int32),
            pltpu.VMEM((bpw,), jnp.int32),
            pltpu.VMEM((CHUNK, 64), jnp.float32),
            pltpu.VMEM((CHUNK, 64), jnp.float32),
            pltpu.VMEM((CHUNK, 64), jnp.float32),
            pltpu.VMEM((CHUNK, 64), jnp.float32),
            pltpu.VMEM((bpw,), jnp.float32),
            pltpu.SemaphoreType.DMA,
            pltpu.SemaphoreType.DMA,
        ],
    )
    def sc_kernel(u_hbm, i_hbm, uvec_hbm, ivec_hbm, out_hbm,
                  uidx_v, iidx_v, ubuf0, ibuf0, ubuf1, ibuf1, out_v,
                  sem0, sem1):
        wid = lax.axis_index("s") * nc + lax.axis_index("c")
        base = wid * bpw
        lane_iota = lax.broadcasted_iota(jnp.int32, (LANES,), 0)

        pltpu.sync_copy(u_hbm.at[pl.ds(base, bpw)], uidx_v)
        pltpu.sync_copy(i_hbm.at[pl.ds(base, bpw)], iidx_v)

        ubufs = (ubuf0, ubuf1)
        ibufs = (ibuf0, ibuf1)
        sems = (sem0, sem1)

        def fire(c, ub, ib, sem):
            def group(g, _):
                uv = uidx_v[pl.ds(c * CHUNK + g * LANES, LANES)]
                iv = iidx_v[pl.ds(c * CHUNK + g * LANES, LANES)]
                for k in range(LANES):
                    m = lane_iota == k
                    ru = jnp.sum(jnp.where(m, uv, 0))
                    ri = jnp.sum(jnp.where(m, iv, 0))
                    r = g * LANES + k
                    pltpu.async_copy(uvec_hbm.at[pl.ds(ru, 1), :],
                                     ub.at[pl.ds(r, 1), :], sem)
                    pltpu.async_copy(ivec_hbm.at[pl.ds(ri, 1), :],
                                     ib.at[pl.ds(r, 1), :], sem)
                return _
            lax.fori_loop(0, GROUPS, group, None)

        def drain(ub, ib, sem):
            def one(r, _):
                pltpu.make_async_copy(uvec_hbm.at[pl.ds(0, 1), :],
                                      ub.at[pl.ds(0, 1), :], sem).wait()
                pltpu.make_async_copy(ivec_hbm.at[pl.ds(0, 1), :],
                                      ib.at[pl.ds(0, 1), :], sem).wait()
                return _
            lax.fori_loop(0, CHUNK, one, None)

        def compute(c, ub, ib):
            def group(g, _):
                rows = g * LANES + lane_iota
                acc = jnp.zeros((LANES,), jnp.float32)
                for f in range(n_factors):
                    cols = jnp.full((LANES,), f, jnp.int32)
                    uvals = plsc.load_gather(ub, [rows, cols])
                    ivals = plsc.load_gather(ib, [rows, cols])
                    acc = acc + uvals * ivals
                out_v[pl.ds(c * CHUNK + g * LANES, LANES)] = acc
                return _
            lax.fori_loop(0, GROUPS, group, None)

        fire(0, ubufs[0], ibufs[0], sems[0])
        for c in range(nch):
            p = c % 2
            if c + 1 < nch:
                fire(c + 1, ubufs[1 - p], ibufs[1 - p], sems[1 - p])
            drain(ubufs[p], ibufs[p], sems[p])
            compute(c, ubufs[p], ibufs[p])

        pltpu.sync_copy(out_v, out_hbm.at[pl.ds(base, bpw)])

    return sc_kernel


def kernel(u, i, user_vec, item_vec):
    batch = u.shape[0]
    n_factors = user_vec.shape[1]
    info = plsc.get_sparse_core_info()
    nc, ns = info.num_cores, info.num_subcores
    n_workers = nc * ns
    fn = _sc_dot_kernel(batch, n_factors, n_workers, nc)
    return fn(u, i, user_vec, item_vec)


# speedup vs baseline: 1.5547x; 1.0046x over previous
import functools

import jax
import jax.numpy as jnp
from jax import lax
from jax.experimental import pallas as pl
from jax.experimental.pallas import tpu as pltpu
from jax.experimental.pallas import tpu_sc as plsc

LANES = 16
CHUNK = 128
GROUPS = CHUNK // LANES


def _sc_dot_kernel(batch, n_factors, n_workers, nc):
    bpw = batch // n_workers
    nch = bpw // CHUNK

    mesh = plsc.VectorSubcoreMesh(core_axis_name="c", subcore_axis_name="s")

    @functools.partial(
        pl.kernel,
        mesh=mesh,
        out_type=jax.ShapeDtypeStruct((batch,), jnp.float32),
        compiler_params=pltpu.CompilerParams(needs_layout_passes=False),
        scratch_types=[
            pltpu.VMEM((bpw,), jnp.int32),
            pltpu.VMEM((bpw,), jnp.int32),
            pltpu.VMEM((CHUNK, 64), jnp.float32),
            pltpu.VMEM((CHUNK, 64), jnp.float32),
            pltpu.VMEM((CHUNK, 64), jnp.float32),
            pltpu.VMEM((CHUNK, 64), jnp.float32),
            pltpu.VMEM((bpw,), jnp.float32),
            pltpu.SemaphoreType.DMA,
            pltpu.SemaphoreType.DMA,
        ],
    )
    def sc_kernel(u_hbm, i_hbm, uvec_hbm, ivec_hbm, out_hbm,
                  uidx_v, iidx_v, ubuf0, ibuf0, ubuf1, ibuf1, out_v,
                  sem0, sem1):
        wid = lax.axis_index("s") * nc + lax.axis_index("c")
        base = wid * bpw
        lane_iota = lax.broadcasted_iota(jnp.int32, (LANES,), 0)

        pltpu.sync_copy(u_hbm.at[pl.ds(base, bpw)], uidx_v)
        pltpu.sync_copy(i_hbm.at[pl.ds(base, bpw)], iidx_v)

        ubufs = (ubuf0, ubuf1)
        ibufs = (ibuf0, ibuf1)
        sems = (sem0, sem1)

        def fire(c, ub, ib, sem):
            def group(g, _):
                uv = uidx_v[pl.ds(c * CHUNK + g * LANES, LANES)]
                iv = iidx_v[pl.ds(c * CHUNK + g * LANES, LANES)]
                for k in range(LANES):
                    m = lane_iota == k
                    ru = jnp.sum(jnp.where(m, uv, 0))
                    ri = jnp.sum(jnp.where(m, iv, 0))
                    r = g * LANES + k
                    pltpu.async_copy(uvec_hbm.at[pl.ds(ru, 1), :],
                                     ub.at[pl.ds(r, 1), :], sem)
                    pltpu.async_copy(ivec_hbm.at[pl.ds(ri, 1), :],
                                     ib.at[pl.ds(r, 1), :], sem)
                return _
            lax.fori_loop(0, GROUPS, group, None)

        def drain(ub, ib, sem):
            pltpu.make_async_copy(uvec_hbm.at[pl.ds(0, CHUNK), :],
                                  ub, sem).wait()
            pltpu.make_async_copy(ivec_hbm.at[pl.ds(0, CHUNK), :],
                                  ib, sem).wait()

        def compute(c, ub, ib):
            def group(g, _):
                rows = g * LANES + lane_iota
                acc = jnp.zeros((LANES,), jnp.float32)
                for f in range(n_factors):
                    cols = jnp.full((LANES,), f, jnp.int32)
                    uvals = plsc.load_gather(ub, [rows, cols])
                    ivals = plsc.load_gather(ib, [rows, cols])
                    acc = acc + uvals * ivals
                out_v[pl.ds(c * CHUNK + g * LANES, LANES)] = acc
                return _
            lax.fori_loop(0, GROUPS, group, None)

        fire(0, ubufs[0], ibufs[0], sems[0])
        for c in range(nch):
            p = c % 2
            if c + 1 < nch:
                fire(c + 1, ubufs[1 - p], ibufs[1 - p], sems[1 - p])
            drain(ubufs[p], ibufs[p], sems[p])
            compute(c, ubufs[p], ibufs[p])

        pltpu.sync_copy(out_v, out_hbm.at[pl.ds(base, bpw)])

    return sc_kernel


def kernel(u, i, user_vec, item_vec):
    batch = u.shape[0]
    n_factors = user_vec.shape[1]
    info = plsc.get_sparse_core_info()
    nc, ns = info.num_cores, info.num_subcores
    n_workers = nc * ns
    fn = _sc_dot_kernel(batch, n_factors, n_workers, nc)
    return fn(u, i, user_vec, item_vec)
